# baseline (speedup 1.0000x reference)
import functools

import jax
import jax.numpy as jnp
from jax import lax
from jax.experimental import pallas as pl
from jax.experimental.pallas import tpu as pltpu
from jax.experimental.pallas import tpu_sc as plsc

_N = 10000
_E = 320000
_G = 64
NC = 2
NS = 16
L = 16
W8 = 8
NW = NC * NS
EPW = _E // NW
NP2 = 10112
NPT = NP2 // NS

_HIGH = lax.Precision.HIGHEST


def _make_agg(D, CH, nbuf):
    K = EPW // CH
    mesh = plsc.VectorSubcoreMesh(core_axis_name="c", subcore_axis_name="s")

    if nbuf == 2:
        scratch = [
            pltpu.VMEM((K, CH), jnp.int32),
            pltpu.VMEM((K, CH), jnp.int32),
        ]
    else:
        scratch = [
            pltpu.VMEM((K, CH), jnp.int32),
            pltpu.VMEM((1, CH), jnp.int32),
            pltpu.VMEM((1, CH), jnp.int32),
            pltpu.SemaphoreType.DMA,
            pltpu.SemaphoreType.DMA,
        ]
    scratch += [pltpu.VMEM((CH, D), jnp.float32) for _ in range(2)]
    scratch += [pltpu.VMEM_SHARED((NP2, D), jnp.float32)]
    scratch += [pltpu.SemaphoreType.DMA for _ in range(2)]

    @functools.partial(
        pl.kernel,
        out_type=jax.ShapeDtypeStruct((NC, NP2, D), jnp.float32),
        mesh=mesh,
        compiler_params=pltpu.CompilerParams(use_tc_tiling_on_sc=(D == 128)),
        scratch_types=scratch,
    )
    def agg(tab, src3, dst3, zrows, out, idx_s, *bufs):
        if nbuf == 2:
            idx_d, rows0, rows1, acc, g0, g1 = bufs
        else:
            db0, db1, sd0, sd1, rows0, rows1, acc, g0, g1 = bufs
        c = lax.axis_index("c")
        s = lax.axis_index("s")
        wid = s * NC + c

        base = s * NPT
        pltpu.sync_copy(zrows, acc.at[pl.ds(base, 8)])
        cur = 8
        while cur < NPT:
            n = min(cur, NPT - cur)
            pltpu.sync_copy(acc.at[pl.ds(base, n)],
                            acc.at[pl.ds(base + cur, n)])
            cur += n
        pltpu.sync_copy(src3.at[wid], idx_s)
        if nbuf == 2:
            pltpu.sync_copy(dst3.at[wid], idx_d)
        plsc.subcore_barrier()

        if nbuf == 2:
            pltpu.async_copy(tab.at[idx_s.at[0]], rows0, g0)
            pltpu.async_copy(tab.at[idx_s.at[1]], rows1, g1)

            def body(i, carry):
                j0 = 2 * i
                j1 = 2 * i + 1
                pltpu.make_async_copy(tab.at[idx_s.at[0]], rows0, g0).wait()
                pltpu.sync_copy(rows0, acc.at[idx_d.at[j0]], add=True)

                @pl.when(j0 + 2 < K)
                def _():
                    pltpu.async_copy(tab.at[idx_s.at[j0 + 2]], rows0, g0)

                pltpu.make_async_copy(tab.at[idx_s.at[0]], rows1, g1).wait()
                pltpu.sync_copy(rows1, acc.at[idx_d.at[j1]], add=True)

                @pl.when(j1 + 2 < K)
                def _():
                    pltpu.async_copy(tab.at[idx_s.at[j1 + 2]], rows1, g1)

                return carry

            lax.fori_loop(0, K // 2, body, 0)
        else:
            pltpu.async_copy(dst3.at[wid, pl.ds(0, 1)], db0, sd0)
            pltpu.async_copy(tab.at[idx_s.at[0]], rows0, g0)
            pltpu.async_copy(dst3.at[wid, pl.ds(1, 1)], db1, sd1)
            pltpu.async_copy(tab.at[idx_s.at[1]], rows1, g1)

            def body(i, carry):
                j0 = 2 * i
                j1 = 2 * i + 1
                pltpu.make_async_copy(dst3.at[wid, pl.ds(0, 1)], db0, sd0).wait()
                pltpu.make_async_copy(tab.at[idx_s.at[0]], rows0, g0).wait()
                pltpu.sync_copy(rows0, acc.at[db0.at[0]], add=True)

                @pl.when(j0 + 2 < K)
                def _():
                    pltpu.async_copy(dst3.at[wid, pl.ds(j0 + 2, 1)], db0, sd0)
                    pltpu.async_copy(tab.at[idx_s.at[j0 + 2]], rows0, g0)

                pltpu.make_async_copy(dst3.at[wid, pl.ds(1, 1)], db1, sd1).wait()
                pltpu.make_async_copy(tab.at[idx_s.at[0]], rows1, g1).wait()
                pltpu.sync_copy(rows1, acc.at[db1.at[0]], add=True)

                @pl.when(j1 + 2 < K)
                def _():
                    pltpu.async_copy(dst3.at[wid, pl.ds(j1 + 2, 1)], db1, sd1)
                    pltpu.async_copy(tab.at[idx_s.at[j1 + 2]], rows1, g1)

                return carry

            lax.fori_loop(0, K // 2, body, 0)

        plsc.subcore_barrier()
        pltpu.sync_copy(acc.at[pl.ds(s * NPT, NPT)],
                        out.at[c, pl.ds(s * NPT, NPT)])

    return agg


CH8 = 125
CH128 = 125
_AGG8 = _make_agg(W8, CH8, 2)
_AGG128 = _make_agg(128, CH128, 1)


def _make_deg(CH):
    K = EPW // CH
    mesh = plsc.VectorSubcoreMesh(core_axis_name="c", subcore_axis_name="s")

    @functools.partial(
        pl.kernel,
        out_type=jax.ShapeDtypeStruct((NC, NP2, W8), jnp.float32),
        mesh=mesh,
        compiler_params=pltpu.CompilerParams(use_tc_tiling_on_sc=False),
        scratch_types=[
            pltpu.VMEM((K, CH), jnp.int32),
            pltpu.VMEM((CH, W8), jnp.float32),
            pltpu.VMEM_SHARED((NP2, W8), jnp.float32),
        ],
    )
    def deg(ones_ch, dst3, zrows, out, idx_d, rows, acc):
        c = lax.axis_index("c")
        s = lax.axis_index("s")
        wid = s * NC + c

        base = s * NPT
        pltpu.sync_copy(zrows, acc.at[pl.ds(base, 8)])
        cur = 8
        while cur < NPT:
            n = min(cur, NPT - cur)
            pltpu.sync_copy(acc.at[pl.ds(base, n)],
                            acc.at[pl.ds(base + cur, n)])
            cur += n
        pltpu.sync_copy(dst3.at[wid], idx_d)
        pltpu.sync_copy(ones_ch, rows)
        plsc.subcore_barrier()

        def body(i, carry):
            pltpu.sync_copy(rows, acc.at[idx_d.at[i]], add=True)
            return carry

        lax.fori_loop(0, K, body, 0)
        plsc.subcore_barrier()
        pltpu.sync_copy(acc.at[pl.ds(s * NPT, NPT)],
                        out.at[c, pl.ds(s * NPT, NPT)])

    return deg


_DEG = _make_deg(CH8)


def _tc_scale(degp, x):
    def body(dp, xr, dinv_o, xp_o):
        d8 = (dp[0] + dp[1])[:_N]
        deg = d8[:, 0:1] + 1.0
        dinv = lax.rsqrt(deg)
        dinv_o[...] = dinv
        xpad = jnp.concatenate(
            [xr[...], jnp.zeros((_N, W8 - xr.shape[1]), jnp.float32)], axis=1)
        xp_o[...] = xpad * dinv

    return pl.pallas_call(
        body,
        out_shape=(jax.ShapeDtypeStruct((_N, 1), jnp.float32),
                   jax.ShapeDtypeStruct((_N, W8), jnp.float32)),
    )(degp, x)


def _tc_layer(p, g_prev, dinv, w, b, d_out):
    def body(pr, gr, dr, wr, br, o):
        a = (pr[0] + pr[1])[:_N] + gr[...]
        dv = dr[...]
        z = jnp.dot(dv * a, wr[...], preferred_element_type=jnp.float32,
                    precision=_HIGH) + br[...]
        o[...] = dv * jnp.maximum(z, 0.0)

    return pl.pallas_call(
        body,
        out_shape=jax.ShapeDtypeStruct((_N, d_out), jnp.float32),
    )(p, g_prev, dinv, w, b)


def _tc_final(p, g_prev, dinv, w, b, wfc, bfc, batch2):
    def body(pr, gr, dr, wr, br, wf, bf, bt, o):
        a = (pr[0] + pr[1])[:_N] + gr[...]
        h = jnp.maximum(
            jnp.dot(dr[...] * a, wr[...], preferred_element_type=jnp.float32,
                    precision=_HIGH) + br[...], 0.0)
        y = jnp.dot(h, wf[...], preferred_element_type=jnp.float32,
                    precision=_HIGH)
        gid = lax.broadcasted_iota(jnp.int32, (1, _G), 1)
        oh = (bt[...] == gid).astype(jnp.float32)
        sums = lax.dot_general(oh, y, (((0,), (0,)), ((), ())),
                               preferred_element_type=jnp.float32,
                               precision=_HIGH)
        counts = jnp.sum(oh, axis=0)[:, None]
        o[...] = sums / jnp.maximum(counts, 1.0) + bf[...]

    return pl.pallas_call(
        body,
        out_shape=jax.ShapeDtypeStruct((_G, wfc.shape[1]), jnp.float32),
    )(p, g_prev, dinv, w, b, wfc, bfc, batch2)


def kernel(x, edge_index, batch, W1, b1, W2, b2, W3, b3, Wfc, bfc):
    src8 = edge_index[0].reshape(NW, EPW // CH8, CH8)
    dst8 = edge_index[1].reshape(NW, EPW // CH8, CH8)
    src128 = edge_index[0].reshape(NW, EPW // CH128, CH128)
    dst128 = edge_index[1].reshape(NW, EPW // CH128, CH128)
    batch2 = batch.reshape(_N, 1)
    W1p = jnp.pad(W1, ((0, W8 - W1.shape[0]), (0, 0)))
    b1r = b1.reshape(1, -1)
    b2r = b2.reshape(1, -1)
    b3r = b3.reshape(1, -1)
    bfcr = bfc.reshape(1, -1)
    z8 = jnp.zeros((8, W8), jnp.float32)
    z128 = jnp.zeros((8, 128), jnp.float32)

    ones_ch = jnp.ones((CH8, W8), jnp.float32)

    pdeg = _DEG(ones_ch, dst8, z8)
    dinv, xp = _tc_scale(pdeg, x)
    p1 = _AGG8(xp, src8, dst8, z8)
    g1 = _tc_layer(p1, xp, dinv, W1p, b1r, 128)

    p2 = _AGG128(g1, src128, dst128, z128)
    g2 = _tc_layer(p2, g1, dinv, W2, b2r, 128)
    p3 = _AGG128(g2, src128, dst128, z128)
    return _tc_final(p3, g2, dinv, W3, b3r, Wfc, bfcr, batch2)


# speedup vs baseline: 2.5328x; 2.5328x over previous
import functools

import jax
import jax.numpy as jnp
from jax import lax
from jax.experimental import pallas as pl
from jax.experimental.pallas import tpu as pltpu
from jax.experimental.pallas import tpu_sc as plsc

_N = 10000
_E = 320000
_G = 64
NC = 2
NS = 16
L = 16
W8 = 8
NW = NC * NS
EPW = _E // NW
NP2 = 10112
NPT = NP2 // NS

_HIGH = lax.Precision.HIGHEST


def _make_agg(D, CH, nbuf):
    K = EPW // CH
    mesh = plsc.VectorSubcoreMesh(core_axis_name="c", subcore_axis_name="s")

    if nbuf == 2:
        scratch = [
            pltpu.VMEM((K, CH), jnp.int32),
            pltpu.VMEM((K, CH), jnp.int32),
        ]
    else:
        scratch = [
            pltpu.VMEM((K, CH), jnp.int32),
            pltpu.VMEM((1, CH), jnp.int32),
            pltpu.VMEM((1, CH), jnp.int32),
            pltpu.SemaphoreType.DMA,
            pltpu.SemaphoreType.DMA,
        ]
    scratch += [pltpu.VMEM((CH, D), jnp.float32) for _ in range(2)]
    scratch += [pltpu.VMEM_SHARED((NP2, D), jnp.float32)]
    scratch += [pltpu.SemaphoreType.DMA for _ in range(2)]

    @functools.partial(
        pl.kernel,
        out_type=jax.ShapeDtypeStruct((NC, NP2, D), jnp.float32),
        mesh=mesh,
        compiler_params=pltpu.CompilerParams(use_tc_tiling_on_sc=(D == 128)),
        scratch_types=scratch,
    )
    def agg(tab, src3, dst3, zrows, out, idx_s, *bufs):
        if nbuf == 2:
            idx_d, rows0, rows1, acc, g0, g1 = bufs
        else:
            db0, db1, sd0, sd1, rows0, rows1, acc, g0, g1 = bufs
        c = lax.axis_index("c")
        s = lax.axis_index("s")
        wid = s * NC + c

        pltpu.sync_copy(zrows, acc.at[pl.ds(s * NPT, NPT)])
        pltpu.sync_copy(src3.at[wid], idx_s)
        if nbuf == 2:
            pltpu.sync_copy(dst3.at[wid], idx_d)
        plsc.subcore_barrier()

        if nbuf == 2:
            pltpu.async_copy(tab.at[idx_s.at[0]], rows0, g0)
            pltpu.async_copy(tab.at[idx_s.at[1]], rows1, g1)

            def body(i, carry):
                j0 = 2 * i
                j1 = 2 * i + 1
                pltpu.make_async_copy(tab.at[idx_s.at[0]], rows0, g0).wait()
                pltpu.sync_copy(rows0, acc.at[idx_d.at[j0]], add=True)

                @pl.when(j0 + 2 < K)
                def _():
                    pltpu.async_copy(tab.at[idx_s.at[j0 + 2]], rows0, g0)

                pltpu.make_async_copy(tab.at[idx_s.at[0]], rows1, g1).wait()
                pltpu.sync_copy(rows1, acc.at[idx_d.at[j1]], add=True)

                @pl.when(j1 + 2 < K)
                def _():
                    pltpu.async_copy(tab.at[idx_s.at[j1 + 2]], rows1, g1)

                return carry

            lax.fori_loop(0, K // 2, body, 0)
        else:
            pltpu.async_copy(dst3.at[wid, pl.ds(0, 1)], db0, sd0)
            pltpu.async_copy(tab.at[idx_s.at[0]], rows0, g0)
            pltpu.async_copy(dst3.at[wid, pl.ds(1, 1)], db1, sd1)
            pltpu.async_copy(tab.at[idx_s.at[1]], rows1, g1)

            def body(i, carry):
                j0 = 2 * i
                j1 = 2 * i + 1
                pltpu.make_async_copy(dst3.at[wid, pl.ds(0, 1)], db0, sd0).wait()
                pltpu.make_async_copy(tab.at[idx_s.at[0]], rows0, g0).wait()
                pltpu.sync_copy(rows0, acc.at[db0.at[0]], add=True)

                @pl.when(j0 + 2 < K)
                def _():
                    pltpu.async_copy(dst3.at[wid, pl.ds(j0 + 2, 1)], db0, sd0)
                    pltpu.async_copy(tab.at[idx_s.at[j0 + 2]], rows0, g0)

                pltpu.make_async_copy(dst3.at[wid, pl.ds(1, 1)], db1, sd1).wait()
                pltpu.make_async_copy(tab.at[idx_s.at[0]], rows1, g1).wait()
                pltpu.sync_copy(rows1, acc.at[db1.at[0]], add=True)

                @pl.when(j1 + 2 < K)
                def _():
                    pltpu.async_copy(dst3.at[wid, pl.ds(j1 + 2, 1)], db1, sd1)
                    pltpu.async_copy(tab.at[idx_s.at[j1 + 2]], rows1, g1)

                return carry

            lax.fori_loop(0, K // 2, body, 0)

        plsc.subcore_barrier()
        pltpu.sync_copy(acc.at[pl.ds(s * NPT, NPT)],
                        out.at[c, pl.ds(s * NPT, NPT)])

    return agg


CH8 = 125
CH128 = 125
_AGG8 = _make_agg(W8, CH8, 2)
_AGG128 = _make_agg(128, CH128, 1)


def _make_deg(CH):
    K = EPW // CH
    mesh = plsc.VectorSubcoreMesh(core_axis_name="c", subcore_axis_name="s")

    @functools.partial(
        pl.kernel,
        out_type=jax.ShapeDtypeStruct((NC, NP2, W8), jnp.float32),
        mesh=mesh,
        compiler_params=pltpu.CompilerParams(use_tc_tiling_on_sc=False),
        scratch_types=[
            pltpu.VMEM((K, CH), jnp.int32),
            pltpu.VMEM((CH, W8), jnp.float32),
            pltpu.VMEM_SHARED((NP2, W8), jnp.float32),
        ],
    )
    def deg(ones_ch, dst3, zrows, out, idx_d, rows, acc):
        c = lax.axis_index("c")
        s = lax.axis_index("s")
        wid = s * NC + c

        pltpu.sync_copy(zrows, acc.at[pl.ds(s * NPT, NPT)])
        pltpu.sync_copy(dst3.at[wid], idx_d)
        pltpu.sync_copy(ones_ch, rows)
        plsc.subcore_barrier()

        def body(i, carry):
            pltpu.sync_copy(rows, acc.at[idx_d.at[i]], add=True)
            return carry

        lax.fori_loop(0, K, body, 0)
        plsc.subcore_barrier()
        pltpu.sync_copy(acc.at[pl.ds(s * NPT, NPT)],
                        out.at[c, pl.ds(s * NPT, NPT)])

    return deg


_DEG = _make_deg(CH8)


def _tc_scale(degp, x):
    def body(dp, xr, dinv_o, xp_o):
        d8 = (dp[0] + dp[1])[:_N]
        deg = d8[:, 0:1] + 1.0
        dinv = lax.rsqrt(deg)
        dinv_o[...] = dinv
        xpad = jnp.concatenate(
            [xr[...], jnp.zeros((_N, W8 - xr.shape[1]), jnp.float32)], axis=1)
        xp_o[...] = xpad * dinv

    return pl.pallas_call(
        body,
        out_shape=(jax.ShapeDtypeStruct((_N, 1), jnp.float32),
                   jax.ShapeDtypeStruct((_N, W8), jnp.float32)),
    )(degp, x)


def _tc_layer(p, g_prev, dinv, w, b, d_out):
    def body(pr, gr, dr, wr, br, o):
        a = (pr[0] + pr[1])[:_N] + gr[...]
        dv = dr[...]
        z = jnp.dot(dv * a, wr[...], preferred_element_type=jnp.float32,
                    precision=_HIGH) + br[...]
        o[...] = dv * jnp.maximum(z, 0.0)

    return pl.pallas_call(
        body,
        out_shape=jax.ShapeDtypeStruct((_N, d_out), jnp.float32),
    )(p, g_prev, dinv, w, b)


def _tc_final(p, g_prev, dinv, w, b, wfc, bfc, batch2):
    def body(pr, gr, dr, wr, br, wf, bf, bt, o):
        a = (pr[0] + pr[1])[:_N] + gr[...]
        h = jnp.maximum(
            jnp.dot(dr[...] * a, wr[...], preferred_element_type=jnp.float32,
                    precision=_HIGH) + br[...], 0.0)
        y = jnp.dot(h, wf[...], preferred_element_type=jnp.float32,
                    precision=_HIGH)
        gid = lax.broadcasted_iota(jnp.int32, (1, _G), 1)
        oh = (bt[...] == gid).astype(jnp.float32)
        sums = lax.dot_general(oh, y, (((0,), (0,)), ((), ())),
                               preferred_element_type=jnp.float32,
                               precision=_HIGH)
        counts = jnp.sum(oh, axis=0)[:, None]
        o[...] = sums / jnp.maximum(counts, 1.0) + bf[...]

    return pl.pallas_call(
        body,
        out_shape=jax.ShapeDtypeStruct((_G, wfc.shape[1]), jnp.float32),
    )(p, g_prev, dinv, w, b, wfc, bfc, batch2)


def kernel(x, edge_index, batch, W1, b1, W2, b2, W3, b3, Wfc, bfc):
    src8 = edge_index[0].reshape(NW, EPW // CH8, CH8)
    dst8 = edge_index[1].reshape(NW, EPW // CH8, CH8)
    src128 = edge_index[0].reshape(NW, EPW // CH128, CH128)
    dst128 = edge_index[1].reshape(NW, EPW // CH128, CH128)
    batch2 = batch.reshape(_N, 1)
    W1p = jnp.pad(W1, ((0, W8 - W1.shape[0]), (0, 0)))
    b1r = b1.reshape(1, -1)
    b2r = b2.reshape(1, -1)
    b3r = b3.reshape(1, -1)
    bfcr = bfc.reshape(1, -1)
    z8 = jnp.zeros((NPT, W8), jnp.float32)
    z128 = jnp.zeros((NPT, 128), jnp.float32)

    ones_ch = jnp.ones((CH8, W8), jnp.float32)

    pdeg = _DEG(ones_ch, dst8, z8)
    dinv, xp = _tc_scale(pdeg, x)
    p1 = _AGG8(xp, src8, dst8, z8)
    g1 = _tc_layer(p1, xp, dinv, W1p, b1r, 128)

    p2 = _AGG128(g1, src128, dst128, z128)
    g2 = _tc_layer(p2, g1, dinv, W2, b2r, 128)
    p3 = _AGG128(g2, src128, dst128, z128)
    return _tc_final(p3, g2, dinv, W3, b3r, Wfc, bfcr, batch2)


# speedup vs baseline: 2.6196x; 1.0343x over previous
import functools

import jax
import jax.numpy as jnp
from jax import lax
from jax.experimental import pallas as pl
from jax.experimental.pallas import tpu as pltpu
from jax.experimental.pallas import tpu_sc as plsc

_N = 10000
_E = 320000
_G = 64
NC = 2
NS = 16
L = 16
W8 = 8
NW = NC * NS
EPW = _E // NW
NP2 = 10112
NPT = NP2 // NS

_HIGH = lax.Precision.HIGHEST


def _make_agg(D, CH, nbuf):
    K = EPW // CH
    mesh = plsc.VectorSubcoreMesh(core_axis_name="c", subcore_axis_name="s")

    if nbuf == 2:
        scratch = [
            pltpu.VMEM((K, CH), jnp.int32),
            pltpu.VMEM((K, CH), jnp.int32),
        ]
    else:
        scratch = [
            pltpu.VMEM((K, CH), jnp.int32),
            pltpu.VMEM((1, CH), jnp.int32),
            pltpu.VMEM((1, CH), jnp.int32),
            pltpu.SemaphoreType.DMA,
            pltpu.SemaphoreType.DMA,
        ]
    scratch += [pltpu.VMEM((CH, D), jnp.float32) for _ in range(2)]
    scratch += [pltpu.VMEM_SHARED((NP2, D), jnp.float32)]
    scratch += [pltpu.SemaphoreType.DMA for _ in range(2)]

    @functools.partial(
        pl.kernel,
        out_type=jax.ShapeDtypeStruct((NC, NP2, D), jnp.float32),
        mesh=mesh,
        compiler_params=pltpu.CompilerParams(use_tc_tiling_on_sc=(D == 128)),
        scratch_types=scratch,
    )
    def agg(tab, src3, dst3, zrows, out, idx_s, *bufs):
        if nbuf == 2:
            idx_d, rows0, rows1, acc, g0, g1 = bufs
        else:
            db0, db1, sd0, sd1, rows0, rows1, acc, g0, g1 = bufs
        c = lax.axis_index("c")
        s = lax.axis_index("s")
        wid = s * NC + c

        pltpu.sync_copy(zrows, acc.at[pl.ds(s * NPT, NPT)])
        pltpu.sync_copy(src3.at[wid], idx_s)
        if nbuf == 2:
            pltpu.sync_copy(dst3.at[wid], idx_d)
        plsc.subcore_barrier()

        if nbuf == 2:
            pltpu.async_copy(tab.at[idx_s.at[0]], rows0, g0)
            pltpu.async_copy(tab.at[idx_s.at[1]], rows1, g1)

            def body(i, carry):
                j0 = 2 * i
                j1 = 2 * i + 1
                pltpu.make_async_copy(tab.at[idx_s.at[0]], rows0, g0).wait()
                pltpu.sync_copy(rows0, acc.at[idx_d.at[j0]], add=True)

                @pl.when(j0 + 2 < K)
                def _():
                    pltpu.async_copy(tab.at[idx_s.at[j0 + 2]], rows0, g0)

                pltpu.make_async_copy(tab.at[idx_s.at[0]], rows1, g1).wait()
                pltpu.sync_copy(rows1, acc.at[idx_d.at[j1]], add=True)

                @pl.when(j1 + 2 < K)
                def _():
                    pltpu.async_copy(tab.at[idx_s.at[j1 + 2]], rows1, g1)

                return carry

            lax.fori_loop(0, K // 2, body, 0)
        else:
            pltpu.async_copy(dst3.at[wid, pl.ds(0, 1)], db0, sd0)
            pltpu.async_copy(tab.at[idx_s.at[0]], rows0, g0)
            pltpu.async_copy(dst3.at[wid, pl.ds(1, 1)], db1, sd1)
            pltpu.async_copy(tab.at[idx_s.at[1]], rows1, g1)

            def body(i, carry):
                j0 = 2 * i
                j1 = 2 * i + 1
                pltpu.make_async_copy(dst3.at[wid, pl.ds(0, 1)], db0, sd0).wait()
                pltpu.make_async_copy(tab.at[idx_s.at[0]], rows0, g0).wait()
                pltpu.sync_copy(rows0, acc.at[db0.at[0]], add=True)

                @pl.when(j0 + 2 < K)
                def _():
                    pltpu.async_copy(dst3.at[wid, pl.ds(j0 + 2, 1)], db0, sd0)
                    pltpu.async_copy(tab.at[idx_s.at[j0 + 2]], rows0, g0)

                pltpu.make_async_copy(dst3.at[wid, pl.ds(1, 1)], db1, sd1).wait()
                pltpu.make_async_copy(tab.at[idx_s.at[0]], rows1, g1).wait()
                pltpu.sync_copy(rows1, acc.at[db1.at[0]], add=True)

                @pl.when(j1 + 2 < K)
                def _():
                    pltpu.async_copy(dst3.at[wid, pl.ds(j1 + 2, 1)], db1, sd1)
                    pltpu.async_copy(tab.at[idx_s.at[j1 + 2]], rows1, g1)

                return carry

            lax.fori_loop(0, K // 2, body, 0)

        plsc.subcore_barrier()
        pltpu.sync_copy(acc.at[pl.ds(s * NPT, NPT)],
                        out.at[c, pl.ds(s * NPT, NPT)])

    return agg


CH8 = 125
CH128 = 125
_AGG8 = _make_agg(W8, CH8, 2)
_AGG128 = _make_agg(128, CH128, 1)


def _make_deg(CH):
    K = EPW // CH
    mesh = plsc.VectorSubcoreMesh(core_axis_name="c", subcore_axis_name="s")

    @functools.partial(
        pl.kernel,
        out_type=jax.ShapeDtypeStruct((NC, NP2, W8), jnp.float32),
        mesh=mesh,
        compiler_params=pltpu.CompilerParams(use_tc_tiling_on_sc=False),
        scratch_types=[
            pltpu.VMEM((K, CH), jnp.int32),
            pltpu.VMEM((CH, W8), jnp.float32),
            pltpu.VMEM_SHARED((NP2, W8), jnp.float32),
        ],
    )
    def deg(ones_ch, dst3, zrows, out, idx_d, rows, acc):
        c = lax.axis_index("c")
        s = lax.axis_index("s")
        wid = s * NC + c

        pltpu.sync_copy(zrows, acc.at[pl.ds(s * NPT, NPT)])
        pltpu.sync_copy(dst3.at[wid], idx_d)
        pltpu.sync_copy(ones_ch, rows)
        plsc.subcore_barrier()

        def body(i, carry):
            pltpu.sync_copy(rows, acc.at[idx_d.at[i]], add=True)
            return carry

        lax.fori_loop(0, K, body, 0)
        plsc.subcore_barrier()
        pltpu.sync_copy(acc.at[pl.ds(s * NPT, NPT)],
                        out.at[c, pl.ds(s * NPT, NPT)])

    return deg


_DEG = _make_deg(CH8)


def _tc_scale(degp, x):
    def body(dp, xr, dinv_o, xp_o):
        d8 = (dp[0] + dp[1])[:_N]
        deg = d8[:, 0:1] + 1.0
        dinv = lax.rsqrt(deg)
        dinv_o[...] = dinv
        xpad = jnp.concatenate(
            [xr[...], jnp.zeros((_N, W8 - xr.shape[1]), jnp.float32)], axis=1)
        xp_o[...] = xpad * dinv

    return pl.pallas_call(
        body,
        out_shape=(jax.ShapeDtypeStruct((_N, 1), jnp.float32),
                   jax.ShapeDtypeStruct((_N, W8), jnp.float32)),
    )(degp, x)


def _tc_layer(p, g_prev, dinv, w, b, d_out):
    def body(pr, gr, dr, wr, br, o):
        a = (pr[0] + pr[1])[:_N] + gr[...]
        dv = dr[...]
        z = jnp.dot(dv * a, wr[...], preferred_element_type=jnp.float32,
                    precision=_HIGH) + br[...]
        o[...] = dv * jnp.maximum(z, 0.0)

    return pl.pallas_call(
        body,
        out_shape=jax.ShapeDtypeStruct((_N, d_out), jnp.float32),
    )(p, g_prev, dinv, w, b)


def _tc_final(p, g_prev, dinv, w, b, wfc, bfc, batch2):
    def body(pr, gr, dr, wr, br, wf, bf, bt, o):
        a = (pr[0] + pr[1])[:_N] + gr[...]
        h = jnp.maximum(
            jnp.dot(dr[...] * a, wr[...], preferred_element_type=jnp.float32,
                    precision=_HIGH) + br[...], 0.0)
        gid = lax.broadcasted_iota(jnp.int32, (1, _G), 1)
        oh = (bt[...] == gid).astype(jnp.float32)
        sums = lax.dot_general(oh, h, (((0,), (0,)), ((), ())),
                               preferred_element_type=jnp.float32,
                               precision=_HIGH)
        counts = jnp.sum(oh, axis=0)[:, None]
        pooled = sums / jnp.maximum(counts, 1.0)
        o[...] = jnp.dot(pooled, wf[...], preferred_element_type=jnp.float32,
                         precision=_HIGH) + bf[...]

    return pl.pallas_call(
        body,
        out_shape=jax.ShapeDtypeStruct((_G, wfc.shape[1]), jnp.float32),
    )(p, g_prev, dinv, w, b, wfc, bfc, batch2)


def kernel(x, edge_index, batch, W1, b1, W2, b2, W3, b3, Wfc, bfc):
    src8 = edge_index[0].reshape(NW, EPW // CH8, CH8)
    dst8 = edge_index[1].reshape(NW, EPW // CH8, CH8)
    src128 = edge_index[0].reshape(NW, EPW // CH128, CH128)
    dst128 = edge_index[1].reshape(NW, EPW // CH128, CH128)
    batch2 = batch.reshape(_N, 1)
    W1p = jnp.pad(W1, ((0, W8 - W1.shape[0]), (0, 0)))
    b1r = b1.reshape(1, -1)
    b2r = b2.reshape(1, -1)
    b3r = b3.reshape(1, -1)
    bfcr = bfc.reshape(1, -1)
    z8 = jnp.zeros((NPT, W8), jnp.float32)
    z128 = jnp.zeros((NPT, 128), jnp.float32)

    ones_ch = jnp.ones((CH8, W8), jnp.float32)

    pdeg = _DEG(ones_ch, dst8, z8)
    dinv, xp = _tc_scale(pdeg, x)
    p1 = _AGG8(xp, src8, dst8, z8)
    g1 = _tc_layer(p1, xp, dinv, W1p, b1r, 128)

    p2 = _AGG128(g1, src128, dst128, z128)
    g2 = _tc_layer(p2, g1, dinv, W2, b2r, 128)
    p3 = _AGG128(g2, src128, dst128, z128)
    return _tc_final(p3, g2, dinv, W3, b3r, Wfc, bfcr, batch2)
